# Initial kernel scaffold; baseline (speedup 1.0000x reference)
import jax
import jax.numpy as jnp
from jax.experimental import pallas as pl


def kernel(x, edge_index, edge_attr, params):
    raise NotImplementedError("write your pallas kernel here")


# speedup vs baseline: 2.3237x; 2.3237x over previous
import functools

import jax
import jax.numpy as jnp
from jax import lax
from jax.experimental import pallas as pl
from jax.experimental.pallas import tpu as pltpu
from jax.experimental.pallas import tpu_sc as plsc

N_N = 10000
N_E = 320000
DL = 16
DO = 4

CHUNK = 128
CPT = 80
EP = 32 * CPT * CHUNK
GRP = 8
HGRP = 40
NPT = N_N // 16

BLK_E = 4096
BLK_N = 2000


def _ln(h, g, b):
    mu = jnp.mean(h, axis=-1, keepdims=True)
    d = h - mu
    var = jnp.mean(d * d, axis=-1, keepdims=True)
    return d / jnp.sqrt(var + 1e-5) * g + b


def _encn_body(x_ref, w1, b1, wh, bh, wo, bo, g, bt, o_ref):
    h = jnp.maximum(x_ref[...] @ w1[...] + b1[...], 0.0)
    h = jnp.maximum(h @ wh[...] + bh[...], 0.0)
    h = h @ wo[...] + bo[...]
    o_ref[...] = _ln(h, g[...], bt[...])


def _ence_body(ea_ref, w1, b1, wh, bh, wo, bo, g, bt, o_ref):
    h = jnp.maximum(ea_ref[...] @ w1[...] + b1[...], 0.0)
    h = jnp.maximum(h @ wh[...] + bh[...], 0.0)
    h = h @ wo[...] + bo[...]
    h = _ln(h, g[...], bt[...])
    rows = pl.program_id(0) * BLK_E + lax.broadcasted_iota(jnp.int32, (BLK_E, 1), 0)
    o_ref[...] = jnp.where(rows < N_E, h, 0.0)


def _emlp_body(he_ref, hs_ref, hd_ref, w1e, w1s, w1d, b1, wh, bh, wo, bo, g, bt,
               o_ref):
    he = he_ref[...]
    h = he @ w1e[...] + hs_ref[...] @ w1s[...] + hd_ref[...] @ w1d[...] + b1[...]
    h = jnp.maximum(h, 0.0)
    h = jnp.maximum(h @ wh[...] + bh[...], 0.0)
    h = h @ wo[...] + bo[...]
    h = _ln(h, g[...], bt[...]) + he
    rows = pl.program_id(0) * BLK_E + lax.broadcasted_iota(jnp.int32, (BLK_E, 1), 0)
    o_ref[...] = jnp.where(rows < N_E, h, 0.0)


def _nmlp_body(hn_ref, p0_ref, p1_ref, w1n, w1p, b1, wh, bh, wo, bo, g, bt,
               o_ref):
    hn = hn_ref[...]
    pe = p0_ref[...] + p1_ref[...]
    h = jnp.maximum(hn @ w1n[...] + pe @ w1p[...] + b1[...], 0.0)
    h = jnp.maximum(h @ wh[...] + bh[...], 0.0)
    h = h @ wo[...] + bo[...]
    o_ref[...] = _ln(h, g[...], bt[...]) + hn


def _dec_body(hn_ref, w1, b1, wh, bh, wo, bo, o_ref):
    h = jnp.maximum(hn_ref[...] @ w1[...] + b1[...], 0.0)
    h = jnp.maximum(h @ wh[...] + bh[...], 0.0)
    o_ref[...] = h @ wo[...] + bo[...]


def _whole(shape):
    nd = len(shape)
    return pl.BlockSpec(shape, lambda i, _nd=nd: (0,) * nd)


def _rows(nrow, ncol):
    return pl.BlockSpec((nrow, ncol), lambda i: (i, 0))


def _tc_call(body, grid, in_specs, out_rows, out_cols, n_rows, n_out=1):
    out_shape = [jax.ShapeDtypeStruct((n_rows, out_cols), jnp.float32)
                 for _ in range(n_out)]
    out_specs = [_rows(out_rows, out_cols) for _ in range(n_out)]
    if n_out == 1:
        out_shape, out_specs = out_shape[0], out_specs[0]
    return pl.pallas_call(
        body,
        grid=(grid,),
        in_specs=in_specs,
        out_specs=out_specs,
        out_shape=out_shape,
        compiler_params=pltpu.CompilerParams(
            dimension_semantics=("arbitrary",)),
    )


def _gather_body(hn_hbm, src_hbm, dst_hbm, os_hbm, od_hbm,
                 idxs, idxd, rows_s, rows_d, sem_s, sem_d):
    cid = lax.axis_index("c")
    sid = lax.axis_index("s")
    wid = sid * 2 + cid
    pltpu.sync_copy(src_hbm.at[pl.ds(wid * CPT, CPT)], idxs)
    pltpu.sync_copy(dst_hbm.at[pl.ds(wid * CPT, CPT)], idxd)

    def g_body(gi, carry):
        cps = []
        for j in range(GRP):
            cps.append(pltpu.async_copy(
                hn_hbm.at[idxs.at[gi * GRP + j]], rows_s.at[j], sem_s))
            cps.append(pltpu.async_copy(
                hn_hbm.at[idxd.at[gi * GRP + j]], rows_d.at[j], sem_d))
        for c in cps:
            c.wait()
        pltpu.sync_copy(rows_s, os_hbm.at[pl.ds(wid * CPT + gi * GRP, GRP)])
        pltpu.sync_copy(rows_d, od_hbm.at[pl.ds(wid * CPT + gi * GRP, GRP)])
        return carry

    lax.fori_loop(0, CPT // GRP, g_body, 0)


def _scatter_body(he_hbm, dst_hbm, out_hbm, idx, he_v, zb, acc, sem):
    cid = lax.axis_index("c")
    sid = lax.axis_index("s")
    wid = cid * 16 + sid

    def z_body(i, carry):
        zb[i, :] = jnp.zeros((DL,), jnp.float32)
        return carry

    lax.fori_loop(0, NPT, z_body, 0)
    pltpu.sync_copy(zb, acc.at[pl.ds(sid * NPT, NPT)])
    plsc.subcore_barrier()

    pltpu.sync_copy(dst_hbm.at[pl.ds(wid * CPT, CPT)], idx)
    for half in range(2):
        pltpu.sync_copy(he_hbm.at[pl.ds(wid * CPT + half * HGRP, HGRP)], he_v)

        def s_body(j, carry, _half=half):
            pltpu.sync_copy(he_v.at[j], acc.at[idx.at[_half * HGRP + j]],
                            add=True)
            return carry

        lax.fori_loop(0, HGRP, s_body, 0)
    plsc.subcore_barrier()
    pltpu.sync_copy(acc.at[pl.ds(sid * NPT, NPT)],
                    out_hbm.at[cid, pl.ds(sid * NPT, NPT)])


@functools.cache
def _sc_calls():
    mesh = plsc.VectorSubcoreMesh(core_axis_name="c", subcore_axis_name="s")
    sc_params = pltpu.CompilerParams(use_tc_tiling_on_sc=False)
    gather = pl.kernel(
        _gather_body,
        compiler_params=sc_params,
        out_type=[jax.ShapeDtypeStruct((EP // CHUNK, CHUNK, DL), jnp.float32)
                  for _ in range(2)],
        mesh=mesh,
        scratch_types=[
            pltpu.VMEM((CPT, CHUNK), jnp.int32),
            pltpu.VMEM((CPT, CHUNK), jnp.int32),
            pltpu.VMEM((GRP, CHUNK, DL), jnp.float32),
            pltpu.VMEM((GRP, CHUNK, DL), jnp.float32),
            pltpu.SemaphoreType.DMA,
            pltpu.SemaphoreType.DMA,
        ],
    )
    scatter = pl.kernel(
        _scatter_body,
        compiler_params=sc_params,
        out_type=jax.ShapeDtypeStruct((2, N_N, DL), jnp.float32),
        mesh=mesh,
        scratch_types=[
            pltpu.VMEM((CPT, CHUNK), jnp.int32),
            pltpu.VMEM((HGRP, CHUNK, DL), jnp.float32),
            pltpu.VMEM((NPT, DL), jnp.float32),
            pltpu.VMEM_SHARED((N_N, DL), jnp.float32),
            pltpu.SemaphoreType.DMA,
        ],
    )
    return gather, scatter


def _mlp_w(p):
    return (p['W_in'], p['b_in'].reshape(1, -1), p['Wh'][0],
            p['bh'][0].reshape(1, -1), p['W_out'], p['b_out'].reshape(1, -1))


def kernel(x, edge_index, edge_attr, params):
    gather, scatter = _sc_calls()

    pad = EP - N_E
    src2 = jnp.concatenate(
        [edge_index[0], jnp.zeros((pad,), jnp.int32)]).reshape(EP // CHUNK, CHUNK)
    dst2 = jnp.concatenate(
        [edge_index[1], jnp.zeros((pad,), jnp.int32)]).reshape(EP // CHUNK, CHUNK)
    ea_p = jnp.pad(edge_attr, ((0, pad), (0, 0)))

    pn = params['enc_n']
    hn = _tc_call(
        _encn_body, N_N // BLK_N,
        [_rows(BLK_N, 128)] + [_whole(w.shape) for w in
                               _mlp_w(pn) + (pn['gamma'].reshape(1, -1),
                                             pn['beta'].reshape(1, -1))],
        BLK_N, DL, N_N,
    )(x, *_mlp_w(pn), pn['gamma'].reshape(1, -1), pn['beta'].reshape(1, -1))

    pe = params['enc_e']
    he = _tc_call(
        _ence_body, EP // BLK_E,
        [_rows(BLK_E, DO)] + [_whole(w.shape) for w in
                              _mlp_w(pe) + (pe['gamma'].reshape(1, -1),
                                            pe['beta'].reshape(1, -1))],
        BLK_E, DL, EP,
    )(ea_p, *_mlp_w(pe), pe['gamma'].reshape(1, -1), pe['beta'].reshape(1, -1))

    for i in range(len(params['proc_e'])):
        hs3, hd3 = gather(hn, src2, dst2)
        hs = hs3.reshape(EP, DL)
        hd = hd3.reshape(EP, DL)

        pp = params['proc_e'][i]
        w1 = pp['W_in']
        ws = (w1[:DL], w1[DL:2 * DL], w1[2 * DL:], pp['b_in'].reshape(1, -1),
              pp['Wh'][0], pp['bh'][0].reshape(1, -1), pp['W_out'],
              pp['b_out'].reshape(1, -1), pp['gamma'].reshape(1, -1),
              pp['beta'].reshape(1, -1))
        he = _tc_call(
            _emlp_body, EP // BLK_E,
            [_rows(BLK_E, DL)] * 3 + [_whole(w.shape) for w in ws],
            BLK_E, DL, EP,
        )(he, hs, hd, *ws)

        parts = scatter(he.reshape(EP // CHUNK, CHUNK, DL), dst2)

        pn_i = params['proc_n'][i]
        w1 = pn_i['W_in']
        ws = (w1[:DL], w1[DL:], pn_i['b_in'].reshape(1, -1),
              pn_i['Wh'][0], pn_i['bh'][0].reshape(1, -1), pn_i['W_out'],
              pn_i['b_out'].reshape(1, -1), pn_i['gamma'].reshape(1, -1),
              pn_i['beta'].reshape(1, -1))
        hn = _tc_call(
            _nmlp_body, N_N // BLK_N,
            [_rows(BLK_N, DL)] * 3 + [_whole(w.shape) for w in ws],
            BLK_N, DL, N_N,
        )(hn, parts[0], parts[1], *ws)

    pd = params['out']
    return _tc_call(
        _dec_body, N_N // BLK_N,
        [_rows(BLK_N, DL)] + [_whole(w.shape) for w in _mlp_w(pd)],
        BLK_N, DO, N_N,
    )(hn, *_mlp_w(pd))


# speedup vs baseline: 4.6738x; 2.0114x over previous
import functools

import jax
import jax.numpy as jnp
from jax import lax
from jax.experimental import pallas as pl
from jax.experimental.pallas import tpu as pltpu
from jax.experimental.pallas import tpu_sc as plsc

N_N = 10000
N_E = 320000
DL = 16
DO = 4

CHUNK = 128
CPT = 80
EP = 32 * CPT * CHUNK
GRP = 8
HGRP = 40
NPT = N_N // 16

BLK_E = 4096
BLK_N = 2000
EP8 = EP // 8
E8 = N_E // 8


def _ln(h, g, b):
    mu = jnp.mean(h, axis=-1, keepdims=True)
    d = h - mu
    var = jnp.mean(d * d, axis=-1, keepdims=True)
    return d / jnp.sqrt(var + 1e-5) * g + b


def _encn_body(x_ref, w1, b1, wh, bh, wo, bo, g, bt, o_ref):
    h = jnp.maximum(x_ref[...] @ w1[...] + b1[...], 0.0)
    h = jnp.maximum(h @ wh[...] + bh[...], 0.0)
    h = h @ wo[...] + bo[...]
    o_ref[...] = _ln(h, g[...], bt[...])


def _hdot(a, b):
    ah = a.astype(jnp.bfloat16)
    al = (a - ah.astype(jnp.float32)).astype(jnp.bfloat16)
    bh = b.astype(jnp.bfloat16)
    bl = (b - bh.astype(jnp.float32)).astype(jnp.bfloat16)
    f32 = jnp.float32
    return (lax.dot(ah, bh, preferred_element_type=f32)
            + lax.dot(ah, bl, preferred_element_type=f32)
            + lax.dot(al, bh, preferred_element_type=f32))


def _ln8(d, m8, g, b):
    s2 = _hdot(d * d, m8)
    return d / jnp.sqrt(s2 + 1e-5) * g + b


def _ence_body(ea_ref, w1, b1, wh, bh, wo, bo, m8, g, bt, o_ref):
    h = jnp.maximum(_hdot(ea_ref[...], w1[...]) + b1[...], 0.0)
    h = jnp.maximum(_hdot(h, wh[...]) + bh[...], 0.0)
    h = _hdot(h, wo[...]) + bo[...]
    h = _ln8(h, m8[...], g[...], bt[...])
    rows = pl.program_id(0) * BLK_E + lax.broadcasted_iota(jnp.int32, (BLK_E, 1), 0)
    o_ref[...] = jnp.where(rows < E8, h, 0.0)


def _emlp_body(he_ref, hs_ref, hd_ref, w1e, w1s, w1d, b1, wh, bh, wo, bo, m8,
               g, bt, o_ref):
    he = he_ref[...]
    h = (_hdot(he, w1e[...]) + _hdot(hs_ref[...], w1s[...])
         + _hdot(hd_ref[...], w1d[...]) + b1[...])
    h = jnp.maximum(h, 0.0)
    h = jnp.maximum(_hdot(h, wh[...]) + bh[...], 0.0)
    h = _hdot(h, wo[...]) + bo[...]
    h = _ln8(h, m8[...], g[...], bt[...]) + he
    rows = pl.program_id(0) * BLK_E + lax.broadcasted_iota(jnp.int32, (BLK_E, 1), 0)
    o_ref[...] = jnp.where(rows < E8, h, 0.0)


def _nmlp_body(hn_ref, p0_ref, p1_ref, w1n, w1p, b1, wh, bh, wo, bo, g, bt,
               o_ref):
    hn = hn_ref[...]
    pe = p0_ref[...] + p1_ref[...]
    h = jnp.maximum(hn @ w1n[...] + pe @ w1p[...] + b1[...], 0.0)
    h = jnp.maximum(h @ wh[...] + bh[...], 0.0)
    h = h @ wo[...] + bo[...]
    o_ref[...] = _ln(h, g[...], bt[...]) + hn


def _dec_body(hn_ref, w1, b1, wh, bh, wo, bo, o_ref):
    h = jnp.maximum(hn_ref[...] @ w1[...] + b1[...], 0.0)
    h = jnp.maximum(h @ wh[...] + bh[...], 0.0)
    o_ref[...] = h @ wo[...] + bo[...]


def _whole(shape):
    nd = len(shape)
    return pl.BlockSpec(shape, lambda i, _nd=nd: (0,) * nd)


def _rows(nrow, ncol):
    return pl.BlockSpec((nrow, ncol), lambda i: (i, 0))


def _tc_call(body, grid, in_specs, out_rows, out_cols, n_rows, n_out=1):
    out_shape = [jax.ShapeDtypeStruct((n_rows, out_cols), jnp.float32)
                 for _ in range(n_out)]
    out_specs = [_rows(out_rows, out_cols) for _ in range(n_out)]
    if n_out == 1:
        out_shape, out_specs = out_shape[0], out_specs[0]
    return pl.pallas_call(
        body,
        grid=(grid,),
        in_specs=in_specs,
        out_specs=out_specs,
        out_shape=out_shape,
        compiler_params=pltpu.CompilerParams(
            dimension_semantics=("arbitrary",)),
    )


def _gather_body(hn_hbm, src_hbm, dst_hbm, os_hbm, od_hbm,
                 idxs, idxd, rows_s, rows_d, sem_s, sem_d):
    cid = lax.axis_index("c")
    sid = lax.axis_index("s")
    wid = sid * 2 + cid
    pltpu.sync_copy(src_hbm.at[pl.ds(wid * CPT, CPT)], idxs)
    pltpu.sync_copy(dst_hbm.at[pl.ds(wid * CPT, CPT)], idxd)

    def g_body(gi, carry):
        cps = []
        for j in range(GRP):
            cps.append(pltpu.async_copy(
                hn_hbm.at[idxs.at[gi * GRP + j]], rows_s.at[j], sem_s))
            cps.append(pltpu.async_copy(
                hn_hbm.at[idxd.at[gi * GRP + j]], rows_d.at[j], sem_d))
        for c in cps:
            c.wait()
        pltpu.sync_copy(rows_s, os_hbm.at[pl.ds(wid * CPT + gi * GRP, GRP)])
        pltpu.sync_copy(rows_d, od_hbm.at[pl.ds(wid * CPT + gi * GRP, GRP)])
        return carry

    lax.fori_loop(0, CPT // GRP, g_body, 0)


def _scatter_body(he_hbm, dst_hbm, out_hbm, idx, he_v, zb, acc, sem):
    cid = lax.axis_index("c")
    sid = lax.axis_index("s")
    wid = cid * 16 + sid

    def z_body(i, carry):
        zb[i, :] = jnp.zeros((DL,), jnp.float32)
        return carry

    lax.fori_loop(0, NPT, z_body, 0)
    pltpu.sync_copy(zb, acc.at[pl.ds(sid * NPT, NPT)])
    plsc.subcore_barrier()

    pltpu.sync_copy(dst_hbm.at[pl.ds(wid * CPT, CPT)], idx)
    for half in range(2):
        pltpu.sync_copy(he_hbm.at[pl.ds(wid * CPT + half * HGRP, HGRP)], he_v)

        def s_body(j, carry, _half=half):
            pltpu.sync_copy(he_v.at[j], acc.at[idx.at[_half * HGRP + j]],
                            add=True)
            return carry

        lax.fori_loop(0, HGRP, s_body, 0)
    plsc.subcore_barrier()
    pltpu.sync_copy(acc.at[pl.ds(sid * NPT, NPT)],
                    out_hbm.at[cid, pl.ds(sid * NPT, NPT)])


@functools.cache
def _sc_calls():
    mesh = plsc.VectorSubcoreMesh(core_axis_name="c", subcore_axis_name="s")
    sc_params = pltpu.CompilerParams(use_tc_tiling_on_sc=False)
    gather = pl.kernel(
        _gather_body,
        compiler_params=sc_params,
        out_type=[jax.ShapeDtypeStruct((EP // CHUNK, CHUNK, DL), jnp.float32)
                  for _ in range(2)],
        mesh=mesh,
        scratch_types=[
            pltpu.VMEM((CPT, CHUNK), jnp.int32),
            pltpu.VMEM((CPT, CHUNK), jnp.int32),
            pltpu.VMEM((GRP, CHUNK, DL), jnp.float32),
            pltpu.VMEM((GRP, CHUNK, DL), jnp.float32),
            pltpu.SemaphoreType.DMA,
            pltpu.SemaphoreType.DMA,
        ],
    )
    scatter = pl.kernel(
        _scatter_body,
        compiler_params=sc_params,
        out_type=jax.ShapeDtypeStruct((2, N_N, DL), jnp.float32),
        mesh=mesh,
        scratch_types=[
            pltpu.VMEM((CPT, CHUNK), jnp.int32),
            pltpu.VMEM((HGRP, CHUNK, DL), jnp.float32),
            pltpu.VMEM((NPT, DL), jnp.float32),
            pltpu.VMEM_SHARED((N_N, DL), jnp.float32),
            pltpu.SemaphoreType.DMA,
        ],
    )
    return gather, scatter


def _mlp_w(p):
    return (p['W_in'], p['b_in'].reshape(1, -1), p['Wh'][0],
            p['bh'][0].reshape(1, -1), p['W_out'], p['b_out'].reshape(1, -1))


def _bd8(w):
    return jnp.kron(jnp.eye(8, dtype=w.dtype), w)


def _t8(v):
    return jnp.tile(v.reshape(1, -1), (1, 8))


def _center(wo, bo):
    c = (jnp.eye(DL, dtype=jnp.float32)
         - jnp.full((DL, DL), 1.0 / DL, jnp.float32))
    return wo @ c, bo.reshape(1, -1) @ c


def kernel(x, edge_index, edge_attr, params):
    gather, scatter = _sc_calls()

    pad = EP - N_E
    src2 = jnp.concatenate(
        [edge_index[0], jnp.zeros((pad,), jnp.int32)]).reshape(EP // CHUNK, CHUNK)
    dst2 = jnp.concatenate(
        [edge_index[1], jnp.zeros((pad,), jnp.int32)]).reshape(EP // CHUNK, CHUNK)
    ea_p = jnp.pad(edge_attr, ((0, pad), (0, 0))).reshape(EP8, 32)
    m8 = jnp.kron(jnp.eye(8, dtype=jnp.float32),
                  jnp.full((DL, DL), 1.0 / DL, jnp.float32))

    pn = params['enc_n']
    hn = _tc_call(
        _encn_body, N_N // BLK_N,
        [_rows(BLK_N, 128)] + [_whole(w.shape) for w in
                               _mlp_w(pn) + (pn['gamma'].reshape(1, -1),
                                             pn['beta'].reshape(1, -1))],
        BLK_N, DL, N_N,
    )(x, *_mlp_w(pn), pn['gamma'].reshape(1, -1), pn['beta'].reshape(1, -1))

    pe = params['enc_e']
    woc, boc = _center(pe['W_out'], pe['b_out'])
    ws_e = (_bd8(pe['W_in']), _t8(pe['b_in']), _bd8(pe['Wh'][0]),
            _t8(pe['bh'][0]), _bd8(woc), jnp.tile(boc, (1, 8)),
            m8, _t8(pe['gamma']), _t8(pe['beta']))
    he = _tc_call(
        _ence_body, EP8 // BLK_E,
        [_rows(BLK_E, 32)] + [_whole(w.shape) for w in ws_e],
        BLK_E, 128, EP8,
    )(ea_p, *ws_e)

    for i in range(len(params['proc_e'])):
        hs3, hd3 = gather(hn, src2, dst2)
        hs = hs3.reshape(EP8, 128)
        hd = hd3.reshape(EP8, 128)

        pp = params['proc_e'][i]
        w1 = pp['W_in']
        woc, boc = _center(pp['W_out'], pp['b_out'])
        ws = (_bd8(w1[:DL]), _bd8(w1[DL:2 * DL]), _bd8(w1[2 * DL:]),
              _t8(pp['b_in']), _bd8(pp['Wh'][0]), _t8(pp['bh'][0]),
              _bd8(woc), jnp.tile(boc, (1, 8)),
              m8, _t8(pp['gamma']), _t8(pp['beta']))
        he = _tc_call(
            _emlp_body, EP8 // BLK_E,
            [_rows(BLK_E, 128)] * 3 + [_whole(w.shape) for w in ws],
            BLK_E, 128, EP8,
        )(he, hs, hd, *ws)

        parts = scatter(he.reshape(EP // CHUNK, CHUNK, DL), dst2)

        pn_i = params['proc_n'][i]
        w1 = pn_i['W_in']
        ws = (w1[:DL], w1[DL:], pn_i['b_in'].reshape(1, -1),
              pn_i['Wh'][0], pn_i['bh'][0].reshape(1, -1), pn_i['W_out'],
              pn_i['b_out'].reshape(1, -1), pn_i['gamma'].reshape(1, -1),
              pn_i['beta'].reshape(1, -1))
        hn = _tc_call(
            _nmlp_body, N_N // BLK_N,
            [_rows(BLK_N, DL)] * 3 + [_whole(w.shape) for w in ws],
            BLK_N, DL, N_N,
        )(hn, parts[0], parts[1], *ws)

    pd = params['out']
    return _tc_call(
        _dec_body, N_N // BLK_N,
        [_rows(BLK_N, DL)] + [_whole(w.shape) for w in _mlp_w(pd)],
        BLK_N, DO, N_N,
    )(hn, *_mlp_w(pd))
